# Initial kernel scaffold; baseline (speedup 1.0000x reference)
import functools

import jax
import jax.numpy as jnp
from jax import lax
from jax.experimental import pallas as pl
from jax.experimental.pallas import tpu as pltpu
from jax.experimental.pallas import tpu_sc as plsc

N = 10000
E = 320000
D = 128
ED = 16
H = 4
NG = 64

NC = 2
NS = 16
NW = NC * NS
EW = E // NW
CH = 80
NCHUNK = EW // CH
RS = N // NS

_mesh = plsc.VectorSubcoreMesh(core_axis_name="c", subcore_axis_name="s")


@functools.partial(
    pl.kernel,
    mesh=_mesh,
    out_type=jax.ShapeDtypeStruct((E, D), jnp.float32),
    scratch_types=[
        pltpu.VMEM((CH,), jnp.int32),
        pltpu.VMEM((CH, D), jnp.float32),
        pltpu.SemaphoreType.DMA,
    ],
)
def _sc_gather(table_hbm, idx_hbm, out_hbm, idx_v, rows_v, sem):
    wid = lax.axis_index("s") * NC + lax.axis_index("c")
    wbase = pl.multiple_of(wid * EW, 8)

    def body(ci, carry):
        base = pl.multiple_of(wbase + ci * CH, 8)
        pltpu.sync_copy(idx_hbm.at[pl.ds(base, CH)], idx_v)
        pltpu.async_copy(table_hbm.at[idx_v], rows_v, sem).wait()
        pltpu.sync_copy(rows_v, out_hbm.at[pl.ds(base, CH)])
        return carry

    lax.fori_loop(0, NCHUNK, body, 0)


@functools.partial(
    pl.kernel,
    mesh=_mesh,
    out_type=(
        jax.ShapeDtypeStruct((NC, N, D), jnp.float32),
        jax.ShapeDtypeStruct((NC, N, 16), jnp.float32),
    ),
    scratch_types=[
        pltpu.VMEM((CH,), jnp.int32),
        pltpu.VMEM((CH, D), jnp.float32),
        pltpu.VMEM((CH, 16), jnp.float32),
        pltpu.VMEM_SHARED((N, D), jnp.float32),
        pltpu.VMEM_SHARED((N, 16), jnp.float32),
    ],
)
def _sc_scatter(v128_hbm, v16_hbm, dst_hbm, z128_hbm, z16_hbm,
                o128_hbm, o16_hbm, idx_v, r128_v, r16_v, acc128, acc16):
    cid = lax.axis_index("c")
    sid = lax.axis_index("s")
    wid = sid * NC + cid
    rb = pl.multiple_of(sid * RS, 8)

    pltpu.sync_copy(z128_hbm.at[pl.ds(rb, RS)], acc128.at[pl.ds(rb, RS)])
    pltpu.sync_copy(z16_hbm.at[pl.ds(rb, RS)], acc16.at[pl.ds(rb, RS)])
    plsc.subcore_barrier()

    wbase = pl.multiple_of(wid * EW, 8)

    def body(ci, carry):
        base = pl.multiple_of(wbase + ci * CH, 8)
        pltpu.sync_copy(dst_hbm.at[pl.ds(base, CH)], idx_v)
        pltpu.sync_copy(v128_hbm.at[pl.ds(base, CH)], r128_v)
        pltpu.sync_copy(v16_hbm.at[pl.ds(base, CH)], r16_v)
        pltpu.sync_copy(r128_v, acc128.at[idx_v], add=True)
        pltpu.sync_copy(r16_v, acc16.at[idx_v], add=True)
        return carry

    lax.fori_loop(0, NCHUNK, body, 0)
    plsc.subcore_barrier()

    pltpu.sync_copy(acc128.at[pl.ds(rb, RS)], o128_hbm.at[cid, pl.ds(rb, RS)])
    pltpu.sync_copy(acc16.at[pl.ds(rb, RS)], o16_hbm.at[cid, pl.ds(rb, RS)])


def _mm_body(x_ref, w_ref, b_ref, o_ref):
    o_ref[...] = (jnp.dot(x_ref[...], w_ref[...],
                          preferred_element_type=jnp.float32) + b_ref[...])


def _tc_linear(x, w, b, bn):
    n = x.shape[0]
    din, dout = w.shape
    return pl.pallas_call(
        _mm_body,
        grid=(n // bn,),
        in_specs=[
            pl.BlockSpec((bn, din), lambda i: (i, 0)),
            pl.BlockSpec((din, dout), lambda i: (0, 0)),
            pl.BlockSpec((1, dout), lambda i: (0, 0)),
        ],
        out_specs=pl.BlockSpec((bn, dout), lambda i: (i, 0)),
        out_shape=jax.ShapeDtypeStruct((n, dout), jnp.float32),
    )(x, w, b.reshape(1, dout))


BE = 2000


def _edge_body(g_ref, ea_ref, we_ref, be_ref, m1_ref, m2_ref,
               o128_ref, o16_ref):
    ep = jnp.dot(ea_ref[...], we_ref[...],
                 preferred_element_type=jnp.float32) + be_ref[...]
    m = jnp.maximum(g_ref[...] + ep, 0.0)
    exb = jnp.exp(jnp.dot(m, m1_ref[...], preferred_element_type=jnp.float32))
    o128_ref[...] = m * exb
    l4 = jnp.dot(m, m2_ref[...], preferred_element_type=jnp.float32)
    lane = lax.broadcasted_iota(jnp.int32, (BE, 16), 1)
    o16_ref[...] = jnp.where(lane < H, jnp.exp(l4), 0.0)


def _tc_edge(g, ea, we, be, m1, m2):
    return pl.pallas_call(
        _edge_body,
        grid=(E // BE,),
        in_specs=[
            pl.BlockSpec((BE, D), lambda i: (i, 0)),
            pl.BlockSpec((BE, ED), lambda i: (i, 0)),
            pl.BlockSpec((ED, D), lambda i: (0, 0)),
            pl.BlockSpec((1, D), lambda i: (0, 0)),
            pl.BlockSpec((D, D), lambda i: (0, 0)),
            pl.BlockSpec((D, 16), lambda i: (0, 0)),
        ],
        out_specs=[
            pl.BlockSpec((BE, D), lambda i: (i, 0)),
            pl.BlockSpec((BE, 16), lambda i: (i, 0)),
        ],
        out_shape=[
            jax.ShapeDtypeStruct((E, D), jnp.float32),
            jax.ShapeDtypeStruct((E, 16), jnp.float32),
        ],
    )(g, ea, we, be.reshape(1, D), m1, m2)


BN = 1000


def _node_body(xp_ref, p0_ref, p1_ref, q0_ref, q1_ref, bat_ref, rep_ref,
               w1_ref, b1_ref, gs_ref, bt_ref, w2_ref, b2_ref, eps_ref,
               h_ref, ps_ref, pc_ref):
    den = jnp.dot(q0_ref[...] + q1_ref[...], rep_ref[...],
                  preferred_element_type=jnp.float32) + 1e-16
    agg = (p0_ref[...] + p1_ref[...]) / den
    h1 = (1.0 + eps_ref[0, 0]) * xp_ref[...] + agg
    h2 = jnp.dot(h1, w1_ref[...], preferred_element_type=jnp.float32) + b1_ref[...]
    h2 = jnp.maximum(gs_ref[...] * h2 + bt_ref[...], 0.0)
    h3 = jnp.dot(h2, w2_ref[...], preferred_element_type=jnp.float32) + b2_ref[...]
    hout = jnp.maximum(h3, 0.0)
    h_ref[...] = hout

    g64 = lax.broadcasted_iota(jnp.int32, (NG, BN), 0)
    mask = (g64 == bat_ref[...]).astype(jnp.float32)
    s = jnp.dot(mask, hout, preferred_element_type=jnp.float32)
    c = jnp.dot(mask, jnp.ones((BN, D), jnp.float32),
                preferred_element_type=jnp.float32)

    i = pl.program_id(0)

    @pl.when(i == 0)
    def _():
        ps_ref[...] = s
        pc_ref[...] = c

    @pl.when(i > 0)
    def _():
        ps_ref[...] += s
        pc_ref[...] += c


def _tc_node(xp, p128, p16, bat2d, rep, w1, b1, gs, bt, w2, b2, epsv):
    return pl.pallas_call(
        _node_body,
        grid=(N // BN,),
        in_specs=[
            pl.BlockSpec((BN, D), lambda i: (i, 0)),
            pl.BlockSpec((BN, D), lambda i: (i, 0)),
            pl.BlockSpec((BN, D), lambda i: (i, 0)),
            pl.BlockSpec((BN, 16), lambda i: (i, 0)),
            pl.BlockSpec((BN, 16), lambda i: (i, 0)),
            pl.BlockSpec((1, BN), lambda i: (0, i)),
            pl.BlockSpec((16, D), lambda i: (0, 0)),
            pl.BlockSpec((D, D), lambda i: (0, 0)),
            pl.BlockSpec((1, D), lambda i: (0, 0)),
            pl.BlockSpec((1, D), lambda i: (0, 0)),
            pl.BlockSpec((1, D), lambda i: (0, 0)),
            pl.BlockSpec((D, D), lambda i: (0, 0)),
            pl.BlockSpec((1, D), lambda i: (0, 0)),
            pl.BlockSpec((1, 1), lambda i: (0, 0)),
        ],
        out_specs=[
            pl.BlockSpec((BN, D), lambda i: (i, 0)),
            pl.BlockSpec((NG, D), lambda i: (0, 0)),
            pl.BlockSpec((NG, D), lambda i: (0, 0)),
        ],
        out_shape=[
            jax.ShapeDtypeStruct((N, D), jnp.float32),
            jax.ShapeDtypeStruct((NG, D), jnp.float32),
            jax.ShapeDtypeStruct((NG, D), jnp.float32),
        ],
    )(xp, p128[0], p128[1], p16[0], p16[1], bat2d, rep,
      w1, b1.reshape(1, D), gs, bt, w2, b2.reshape(1, D), epsv)


def _head_body(ps0_ref, ps1_ref, ps2_ref, ps3_ref, ps4_ref, pc_ref,
               w1r_ref, bl1_ref, w2p_ref, bl2_ref, o_ref):
    cnt = jnp.maximum(pc_ref[...], 1.0)
    acc = jnp.broadcast_to(bl1_ref[...], (NG, 5 * D))
    for l, ps in enumerate((ps0_ref, ps1_ref, ps2_ref, ps3_ref, ps4_ref)):
        pooled = ps[...] / cnt
        acc = acc + jnp.dot(pooled, w1r_ref[l],
                            preferred_element_type=jnp.float32)
    r = jnp.maximum(acc, 0.0)
    o_ref[...] = jnp.dot(r, w2p_ref[...],
                         preferred_element_type=jnp.float32) + bl2_ref[0, 0]


def _tc_head(psums, pcnt, w1r, bl1, w2p, bl2):
    return pl.pallas_call(
        _head_body,
        out_shape=jax.ShapeDtypeStruct((NG, D), jnp.float32),
    )(psums[0], psums[1], psums[2], psums[3], psums[4], pcnt,
      w1r, bl1.reshape(1, 5 * D), w2p, bl2.reshape(1, 1))


def kernel(x, edge_index, edge_attr, batch, params, Wl1, bl1, Wl2, bl2):
    assert x.shape == (N, D) and edge_index.shape == (2, E)
    src = edge_index[0].astype(jnp.int32)
    dst = edge_index[1].astype(jnp.int32)
    bat2d = batch.astype(jnp.int32).reshape(1, N)
    ea = edge_attr.astype(jnp.float32)

    z128 = jnp.zeros((N, D), jnp.float32)
    z16 = jnp.zeros((N, 16), jnp.float32)
    eye4 = jnp.eye(H, dtype=jnp.float32)
    rrep = jnp.repeat(eye4, D // H, axis=0)
    rep = jnp.pad(rrep.T, ((0, 12), (0, 0)))

    h = x
    psums = []
    pcnt = None
    for p in params:
        att = p['att']
        ahat = (eye4[:, None, :] * att[:, :, None]).reshape(D, H)
        m1 = ahat @ rrep.T
        m2 = jnp.pad(ahat, ((0, 0), (0, 12)))
        gs = (p['gamma'] / jnp.sqrt(1.0 + 1e-5)).reshape(1, D)
        bt = p['beta'].reshape(1, D)
        epsv = p['eps'].reshape(1, 1)

        xp = _tc_linear(h, p['Wx'], p['bx'], BN)
        g = _sc_gather(xp, src)
        v128, v16 = _tc_edge(g, ea, p['We'], p['be'], m1, m2)
        p128, p16 = _sc_scatter(v128, v16, dst, z128, z16)
        h, ps, pc = _tc_node(xp, p128, p16, bat2d, rep,
                             p['W1'], p['b1'], gs, bt, p['W2'], p['b2'], epsv)
        psums.append(ps)
        if pcnt is None:
            pcnt = pc

    w1r = Wl1.reshape(5, D, 5 * D)
    w2p = jnp.pad(Wl2, ((0, 0), (0, D - 1)))
    out2d = _tc_head(psums, pcnt, w1r, bl1, w2p, bl2)
    return out2d[:, 0]


# speedup vs baseline: 4.6268x; 4.6268x over previous
import functools

import jax
import jax.numpy as jnp
from jax import lax
from jax.experimental import pallas as pl
from jax.experimental.pallas import tpu as pltpu
from jax.experimental.pallas import tpu_sc as plsc

N = 10000
E = 320000
D = 128
ED = 16
H = 4
NG = 64

NC = 2
NS = 16
NW = NC * NS
EW = E // NW
CH = 80
NCHUNK = EW // CH
RS = 624
RTAIL = N - NS * RS

@functools.cache
def _sc_kernels():
    mesh = plsc.VectorSubcoreMesh(core_axis_name="c", subcore_axis_name="s")

    @functools.partial(
        pl.kernel,
        mesh=mesh,
        out_type=jax.ShapeDtypeStruct((E, D), jnp.float32),
        scratch_types=[
            pltpu.VMEM((CH,), jnp.int32),
            pltpu.VMEM((CH, D), jnp.float32),
            pltpu.SemaphoreType.DMA,
        ],
    )
    def _sc_gather(table_hbm, idx_hbm, out_hbm, idx_v, rows_v, sem):
        wid = lax.axis_index("s") * NC + lax.axis_index("c")
        wbase = pl.multiple_of(wid * EW, 8)

        def body(ci, carry):
            base = pl.multiple_of(wbase + ci * CH, 8)
            pltpu.sync_copy(idx_hbm.at[pl.ds(base, CH)], idx_v)
            pltpu.async_copy(table_hbm.at[idx_v], rows_v, sem).wait()
            pltpu.sync_copy(rows_v, out_hbm.at[pl.ds(base, CH)])
            return carry

        lax.fori_loop(0, NCHUNK, body, 0)

    @functools.partial(
        pl.kernel,
        mesh=mesh,
        out_type=jax.ShapeDtypeStruct((NC, N, D), jnp.float32),
        scratch_types=[
            pltpu.VMEM((CH,), jnp.int32),
            pltpu.VMEM((CH, D), jnp.float32),
            pltpu.VMEM_SHARED((N, D), jnp.float32),
        ],
    )
    def _sc_scatter(v_hbm, dst_hbm, z_hbm, o_hbm, idx_v, r_v, acc):
        cid = lax.axis_index("c")
        sid = lax.axis_index("s")
        wid = sid * NC + cid
        rb = pl.multiple_of(sid * RS, 8)

        pltpu.sync_copy(z_hbm.at[pl.ds(rb, RS)], acc.at[pl.ds(rb, RS)])

        @pl.when(sid == 0)
        def _():
            tb = NS * RS
            pltpu.sync_copy(z_hbm.at[pl.ds(tb, RTAIL)], acc.at[pl.ds(tb, RTAIL)])

        plsc.subcore_barrier()

        wbase = pl.multiple_of(wid * EW, 8)

        def body(ci, carry):
            base = pl.multiple_of(wbase + ci * CH, 8)
            pltpu.sync_copy(dst_hbm.at[pl.ds(base, CH)], idx_v)
            pltpu.sync_copy(v_hbm.at[pl.ds(base, CH)], r_v)
            pltpu.sync_copy(r_v, acc.at[idx_v], add=True)
            return carry

        lax.fori_loop(0, NCHUNK, body, 0)
        plsc.subcore_barrier()

        pltpu.sync_copy(acc.at[pl.ds(rb, RS)], o_hbm.at[cid, pl.ds(rb, RS)])

        @pl.when(sid == 0)
        def _():
            tb = NS * RS
            pltpu.sync_copy(acc.at[pl.ds(tb, RTAIL)],
                            o_hbm.at[cid, pl.ds(tb, RTAIL)])

    return _sc_gather, _sc_scatter


def _mm_body(x_ref, w_ref, b_ref, o_ref):
    o_ref[...] = (jnp.dot(x_ref[...], w_ref[...],
                          preferred_element_type=jnp.float32) + b_ref[...])


def _tc_linear(x, w, b, bn):
    n = x.shape[0]
    din, dout = w.shape
    return pl.pallas_call(
        _mm_body,
        grid=(n // bn,),
        in_specs=[
            pl.BlockSpec((bn, din), lambda i: (i, 0)),
            pl.BlockSpec((din, dout), lambda i: (0, 0)),
            pl.BlockSpec((1, dout), lambda i: (0, 0)),
        ],
        out_specs=pl.BlockSpec((bn, dout), lambda i: (i, 0)),
        out_shape=jax.ShapeDtypeStruct((n, dout), jnp.float32),
    )(x, w, b.reshape(1, dout))


BE = 2000


def _edge_body(g_ref, ea_ref, we_ref, be_ref, m1_ref, o128_ref, oex_ref):
    ep = jnp.dot(ea_ref[...], we_ref[...],
                 preferred_element_type=jnp.float32) + be_ref[...]
    m = jnp.maximum(g_ref[...] + ep, 0.0)
    exb = jnp.exp(jnp.dot(m, m1_ref[...], preferred_element_type=jnp.float32))
    o128_ref[...] = m * exb
    oex_ref[...] = exb


def _tc_edge(g, ea, we, be, m1):
    return pl.pallas_call(
        _edge_body,
        grid=(E // BE,),
        in_specs=[
            pl.BlockSpec((BE, D), lambda i: (i, 0)),
            pl.BlockSpec((BE, ED), lambda i: (i, 0)),
            pl.BlockSpec((ED, D), lambda i: (0, 0)),
            pl.BlockSpec((1, D), lambda i: (0, 0)),
            pl.BlockSpec((D, D), lambda i: (0, 0)),
        ],
        out_specs=[
            pl.BlockSpec((BE, D), lambda i: (i, 0)),
            pl.BlockSpec((BE, D), lambda i: (i, 0)),
        ],
        out_shape=[
            jax.ShapeDtypeStruct((E, D), jnp.float32),
            jax.ShapeDtypeStruct((E, D), jnp.float32),
        ],
    )(g, ea, we, be.reshape(1, D), m1)


BN = 1000


def _node_body(xp_ref, p0_ref, p1_ref, q0_ref, q1_ref, bat_ref,
               w1_ref, b1_ref, gs_ref, bt_ref, w2_ref, b2_ref, eps_ref,
               h_ref, ps_ref, pc_ref):
    den = q0_ref[...] + q1_ref[...] + 1e-16
    agg = (p0_ref[...] + p1_ref[...]) / den
    h1 = (1.0 + eps_ref[0, 0]) * xp_ref[...] + agg
    h2 = jnp.dot(h1, w1_ref[...], preferred_element_type=jnp.float32) + b1_ref[...]
    h2 = jnp.maximum(gs_ref[...] * h2 + bt_ref[...], 0.0)
    h3 = jnp.dot(h2, w2_ref[...], preferred_element_type=jnp.float32) + b2_ref[...]
    hout = jnp.maximum(h3, 0.0)
    h_ref[...] = hout

    g64 = lax.broadcasted_iota(jnp.int32, (NG, BN), 0)
    mask = (g64 == bat_ref[0]).astype(jnp.float32)
    s = jnp.dot(mask, hout, preferred_element_type=jnp.float32)
    c = jnp.dot(mask, jnp.ones((BN, D), jnp.float32),
                preferred_element_type=jnp.float32)

    i = pl.program_id(0)

    @pl.when(i == 0)
    def _():
        ps_ref[...] = s
        pc_ref[...] = c

    @pl.when(i > 0)
    def _():
        ps_ref[...] += s
        pc_ref[...] += c


def _tc_node(xp, p128, pex, bat3d, w1, b1, gs, bt, w2, b2, epsv):
    return pl.pallas_call(
        _node_body,
        grid=(N // BN,),
        in_specs=[
            pl.BlockSpec((BN, D), lambda i: (i, 0)),
            pl.BlockSpec((BN, D), lambda i: (i, 0)),
            pl.BlockSpec((BN, D), lambda i: (i, 0)),
            pl.BlockSpec((BN, D), lambda i: (i, 0)),
            pl.BlockSpec((BN, D), lambda i: (i, 0)),
            pl.BlockSpec((1, 1, BN), lambda i: (i, 0, 0)),
            pl.BlockSpec((D, D), lambda i: (0, 0)),
            pl.BlockSpec((1, D), lambda i: (0, 0)),
            pl.BlockSpec((1, D), lambda i: (0, 0)),
            pl.BlockSpec((1, D), lambda i: (0, 0)),
            pl.BlockSpec((D, D), lambda i: (0, 0)),
            pl.BlockSpec((1, D), lambda i: (0, 0)),
            pl.BlockSpec((1, 1), lambda i: (0, 0)),
        ],
        out_specs=[
            pl.BlockSpec((BN, D), lambda i: (i, 0)),
            pl.BlockSpec((NG, D), lambda i: (0, 0)),
            pl.BlockSpec((NG, D), lambda i: (0, 0)),
        ],
        out_shape=[
            jax.ShapeDtypeStruct((N, D), jnp.float32),
            jax.ShapeDtypeStruct((NG, D), jnp.float32),
            jax.ShapeDtypeStruct((NG, D), jnp.float32),
        ],
    )(xp, p128[0], p128[1], pex[0], pex[1], bat3d,
      w1, b1.reshape(1, D), gs, bt, w2, b2.reshape(1, D), epsv)


def _head_body(ps0_ref, ps1_ref, ps2_ref, ps3_ref, ps4_ref, pc_ref,
               w1r_ref, bl1_ref, w2p_ref, bl2_ref, o_ref):
    cnt = jnp.maximum(pc_ref[...], 1.0)
    acc = jnp.broadcast_to(bl1_ref[...], (NG, 5 * D))
    for l, ps in enumerate((ps0_ref, ps1_ref, ps2_ref, ps3_ref, ps4_ref)):
        pooled = ps[...] / cnt
        acc = acc + jnp.dot(pooled, w1r_ref[l],
                            preferred_element_type=jnp.float32)
    r = jnp.maximum(acc, 0.0)
    o_ref[...] = jnp.dot(r, w2p_ref[...],
                         preferred_element_type=jnp.float32) + bl2_ref[0, 0]


def _tc_head(psums, pcnt, w1r, bl1, w2p, bl2):
    return pl.pallas_call(
        _head_body,
        out_shape=jax.ShapeDtypeStruct((NG, D), jnp.float32),
    )(psums[0], psums[1], psums[2], psums[3], psums[4], pcnt,
      w1r, bl1.reshape(1, 5 * D), w2p, bl2.reshape(1, 1))


def kernel(x, edge_index, edge_attr, batch, params, Wl1, bl1, Wl2, bl2):
    assert x.shape == (N, D) and edge_index.shape == (2, E)
    src = edge_index[0].astype(jnp.int32)
    dst = edge_index[1].astype(jnp.int32)
    bat3d = batch.astype(jnp.int32).reshape(N // BN, 1, BN)
    ea = edge_attr.astype(jnp.float32)

    z128 = jnp.zeros((N, D), jnp.float32)
    eye4 = jnp.eye(H, dtype=jnp.float32)
    rrep = jnp.repeat(eye4, D // H, axis=0)

    h = x
    psums = []
    pcnt = None
    for p in params:
        att = p['att']
        ahat = (eye4[:, None, :] * att[:, :, None]).reshape(D, H)
        m1 = ahat @ rrep.T
        gs = (p['gamma'] / jnp.sqrt(1.0 + 1e-5)).reshape(1, D)
        bt = p['beta'].reshape(1, D)
        epsv = p['eps'].reshape(1, 1)

        sc_gather, sc_scatter = _sc_kernels()
        xp = _tc_linear(h, p['Wx'], p['bx'], BN)
        g = sc_gather(xp, src)
        v128, vex = _tc_edge(g, ea, p['We'], p['be'], m1)
        p128 = sc_scatter(v128, dst, z128)
        pex = sc_scatter(vex, dst, z128)
        h, ps, pc = _tc_node(xp, p128, pex, bat3d,
                             p['W1'], p['b1'], gs, bt, p['W2'], p['b2'], epsv)
        psums.append(ps)
        if pcnt is None:
            pcnt = pc

    w1r = Wl1.reshape(5, D, 5 * D)
    w2p = jnp.pad(Wl2, ((0, 0), (0, D - 1)))
    out2d = _tc_head(psums, pcnt, w1r, bl1, w2p, bl2)
    return out2d[:, 0]


# speedup vs baseline: 6.6515x; 1.4376x over previous
import functools

import jax
import jax.numpy as jnp
from jax import lax
from jax.experimental import pallas as pl
from jax.experimental.pallas import tpu as pltpu
from jax.experimental.pallas import tpu_sc as plsc

N = 10000
E = 320000
D = 128
ED = 16
H = 4
NG = 64

NC = 2
NS = 16
NW = NC * NS
EW = E // NW
CH = 80
KG = 5
GE = KG * CH
GPW = EW // GE
CHS = 40
GES = KG * CHS
GPWS = EW // GES
RS = 624
RTAIL = N - NS * RS

@functools.cache
def _sc_kernels():
    mesh = plsc.VectorSubcoreMesh(core_axis_name="c", subcore_axis_name="s")

    @functools.partial(
        pl.kernel,
        mesh=mesh,
        out_type=jax.ShapeDtypeStruct((E, D), jnp.float32),
        scratch_types=[
            pltpu.VMEM((CH,), jnp.int32),
            pltpu.VMEM((CH,), jnp.int32),
            pltpu.VMEM((CH,), jnp.int32),
            pltpu.VMEM((CH,), jnp.int32),
            pltpu.VMEM((CH,), jnp.int32),
            pltpu.VMEM((GE, D), jnp.float32),
            pltpu.SemaphoreType.DMA,
            pltpu.SemaphoreType.DMA,
        ],
    )
    def _sc_gather(table_hbm, idx_hbm, out_hbm, i0, i1, i2, i3, i4,
                   rows_v, seml, semg):
        wid = lax.axis_index("s") * NC + lax.axis_index("c")
        ebase = wid * EW
        ibufs = (i0, i1, i2, i3, i4)

        def body(g, carry):
            base = pl.multiple_of(ebase + g * GE, 8)
            hs = [pltpu.async_copy(
                idx_hbm.at[pl.ds(base + k * CH, CH)], ibufs[k], seml)
                for k in range(KG)]
            for h in hs:
                h.wait()
            gs = [pltpu.async_copy(
                table_hbm.at[ibufs[k]],
                rows_v.at[pl.ds(k * CH, CH)], semg) for k in range(KG)]
            for h in gs:
                h.wait()
            pltpu.sync_copy(rows_v, out_hbm.at[pl.ds(base, GE)])
            return carry

        lax.fori_loop(0, GPW, body, 0)

    @functools.partial(
        pl.kernel,
        mesh=mesh,
        out_type=jax.ShapeDtypeStruct((NC, N, D), jnp.float32),
        scratch_types=[
            pltpu.VMEM((CHS,), jnp.int32),
            pltpu.VMEM((CHS,), jnp.int32),
            pltpu.VMEM((CHS,), jnp.int32),
            pltpu.VMEM((CHS,), jnp.int32),
            pltpu.VMEM((CHS,), jnp.int32),
            pltpu.VMEM((GES, D), jnp.float32),
            pltpu.VMEM_SHARED((N, D), jnp.float32),
            pltpu.SemaphoreType.DMA,
            pltpu.SemaphoreType.DMA,
        ],
    )
    def _sc_scatter(v_hbm, dst_hbm, z_hbm, o_hbm, i0, i1, i2, i3, i4,
                    r_v, acc, seml, sems):
        cid = lax.axis_index("c")
        sid = lax.axis_index("s")
        wid = sid * NC + cid
        rb = pl.multiple_of(sid * RS, 8)
        ibufs = (i0, i1, i2, i3, i4)

        pltpu.sync_copy(z_hbm.at[pl.ds(rb, RS)], acc.at[pl.ds(rb, RS)])

        @pl.when(sid == 0)
        def _():
            tb = NS * RS
            pltpu.sync_copy(z_hbm.at[pl.ds(tb, RTAIL)], acc.at[pl.ds(tb, RTAIL)])

        plsc.subcore_barrier()

        ebase = wid * EW

        def body(g, carry):
            base = pl.multiple_of(ebase + g * GES, 8)
            hs = [pltpu.async_copy(
                dst_hbm.at[pl.ds(base + k * CHS, CHS)], ibufs[k], seml)
                for k in range(KG)]
            hs.append(pltpu.async_copy(
                v_hbm.at[pl.ds(base, GES)], r_v, seml))
            for h in hs:
                h.wait()
            ss = [pltpu.async_copy(
                r_v.at[pl.ds(k * CHS, CHS)],
                acc.at[ibufs[k]], sems, add=True) for k in range(KG)]
            for h in ss:
                h.wait()
            return carry

        lax.fori_loop(0, GPWS, body, 0)
        plsc.subcore_barrier()

        pltpu.sync_copy(acc.at[pl.ds(rb, RS)], o_hbm.at[cid, pl.ds(rb, RS)])

        @pl.when(sid == 0)
        def _():
            tb = NS * RS
            pltpu.sync_copy(acc.at[pl.ds(tb, RTAIL)],
                            o_hbm.at[cid, pl.ds(tb, RTAIL)])

    return _sc_gather, _sc_scatter


def _mm_body(x_ref, w_ref, b_ref, o_ref):
    o_ref[...] = (jnp.dot(x_ref[...], w_ref[...],
                          preferred_element_type=jnp.float32) + b_ref[...])


def _tc_linear(x, w, b, bn):
    n = x.shape[0]
    din, dout = w.shape
    return pl.pallas_call(
        _mm_body,
        grid=(n // bn,),
        in_specs=[
            pl.BlockSpec((bn, din), lambda i: (i, 0)),
            pl.BlockSpec((din, dout), lambda i: (0, 0)),
            pl.BlockSpec((1, dout), lambda i: (0, 0)),
        ],
        out_specs=pl.BlockSpec((bn, dout), lambda i: (i, 0)),
        out_shape=jax.ShapeDtypeStruct((n, dout), jnp.float32),
    )(x, w, b.reshape(1, dout))


BE = 2000


def _edge_body(g_ref, ea_ref, we_ref, be_ref, m1_ref, o128_ref, oex_ref):
    ep = jnp.dot(ea_ref[...], we_ref[...],
                 preferred_element_type=jnp.float32) + be_ref[...]
    m = jnp.maximum(g_ref[...] + ep, 0.0)
    exb = jnp.exp(jnp.dot(m, m1_ref[...], preferred_element_type=jnp.float32))
    o128_ref[...] = m * exb
    oex_ref[...] = exb


def _tc_edge(g, ea, we, be, m1):
    return pl.pallas_call(
        _edge_body,
        grid=(E // BE,),
        in_specs=[
            pl.BlockSpec((BE, D), lambda i: (i, 0)),
            pl.BlockSpec((BE, ED), lambda i: (i, 0)),
            pl.BlockSpec((ED, D), lambda i: (0, 0)),
            pl.BlockSpec((1, D), lambda i: (0, 0)),
            pl.BlockSpec((D, D), lambda i: (0, 0)),
        ],
        out_specs=[
            pl.BlockSpec((BE, D), lambda i: (i, 0)),
            pl.BlockSpec((BE, D), lambda i: (i, 0)),
        ],
        out_shape=[
            jax.ShapeDtypeStruct((E, D), jnp.float32),
            jax.ShapeDtypeStruct((E, D), jnp.float32),
        ],
    )(g, ea, we, be.reshape(1, D), m1)


BN = 1000


def _node_body(xp_ref, p0_ref, p1_ref, q0_ref, q1_ref, bat_ref,
               w1_ref, b1_ref, gs_ref, bt_ref, w2_ref, b2_ref, eps_ref,
               h_ref, ps_ref, pc_ref):
    den = q0_ref[...] + q1_ref[...] + 1e-16
    agg = (p0_ref[...] + p1_ref[...]) / den
    h1 = (1.0 + eps_ref[0, 0]) * xp_ref[...] + agg
    h2 = jnp.dot(h1, w1_ref[...], preferred_element_type=jnp.float32) + b1_ref[...]
    h2 = jnp.maximum(gs_ref[...] * h2 + bt_ref[...], 0.0)
    h3 = jnp.dot(h2, w2_ref[...], preferred_element_type=jnp.float32) + b2_ref[...]
    hout = jnp.maximum(h3, 0.0)
    h_ref[...] = hout

    g64 = lax.broadcasted_iota(jnp.int32, (NG, BN), 0)
    mask = (g64 == bat_ref[0]).astype(jnp.float32)
    s = jnp.dot(mask, hout, preferred_element_type=jnp.float32)
    c = jnp.dot(mask, jnp.ones((BN, D), jnp.float32),
                preferred_element_type=jnp.float32)

    i = pl.program_id(0)

    @pl.when(i == 0)
    def _():
        ps_ref[...] = s
        pc_ref[...] = c

    @pl.when(i > 0)
    def _():
        ps_ref[...] += s
        pc_ref[...] += c


def _tc_node(xp, p128, pex, bat3d, w1, b1, gs, bt, w2, b2, epsv):
    return pl.pallas_call(
        _node_body,
        grid=(N // BN,),
        in_specs=[
            pl.BlockSpec((BN, D), lambda i: (i, 0)),
            pl.BlockSpec((BN, D), lambda i: (i, 0)),
            pl.BlockSpec((BN, D), lambda i: (i, 0)),
            pl.BlockSpec((BN, D), lambda i: (i, 0)),
            pl.BlockSpec((BN, D), lambda i: (i, 0)),
            pl.BlockSpec((1, 1, BN), lambda i: (i, 0, 0)),
            pl.BlockSpec((D, D), lambda i: (0, 0)),
            pl.BlockSpec((1, D), lambda i: (0, 0)),
            pl.BlockSpec((1, D), lambda i: (0, 0)),
            pl.BlockSpec((1, D), lambda i: (0, 0)),
            pl.BlockSpec((D, D), lambda i: (0, 0)),
            pl.BlockSpec((1, D), lambda i: (0, 0)),
            pl.BlockSpec((1, 1), lambda i: (0, 0)),
        ],
        out_specs=[
            pl.BlockSpec((BN, D), lambda i: (i, 0)),
            pl.BlockSpec((NG, D), lambda i: (0, 0)),
            pl.BlockSpec((NG, D), lambda i: (0, 0)),
        ],
        out_shape=[
            jax.ShapeDtypeStruct((N, D), jnp.float32),
            jax.ShapeDtypeStruct((NG, D), jnp.float32),
            jax.ShapeDtypeStruct((NG, D), jnp.float32),
        ],
    )(xp, p128[0], p128[1], pex[0], pex[1], bat3d,
      w1, b1.reshape(1, D), gs, bt, w2, b2.reshape(1, D), epsv)


def _head_body(ps0_ref, ps1_ref, ps2_ref, ps3_ref, ps4_ref, pc_ref,
               w1r_ref, bl1_ref, w2p_ref, bl2_ref, o_ref):
    cnt = jnp.maximum(pc_ref[...], 1.0)
    acc = jnp.broadcast_to(bl1_ref[...], (NG, 5 * D))
    for l, ps in enumerate((ps0_ref, ps1_ref, ps2_ref, ps3_ref, ps4_ref)):
        pooled = ps[...] / cnt
        acc = acc + jnp.dot(pooled, w1r_ref[l],
                            preferred_element_type=jnp.float32)
    r = jnp.maximum(acc, 0.0)
    o_ref[...] = jnp.dot(r, w2p_ref[...],
                         preferred_element_type=jnp.float32) + bl2_ref[0, 0]


def _tc_head(psums, pcnt, w1r, bl1, w2p, bl2):
    return pl.pallas_call(
        _head_body,
        out_shape=jax.ShapeDtypeStruct((NG, D), jnp.float32),
    )(psums[0], psums[1], psums[2], psums[3], psums[4], pcnt,
      w1r, bl1.reshape(1, 5 * D), w2p, bl2.reshape(1, 1))


def kernel(x, edge_index, edge_attr, batch, params, Wl1, bl1, Wl2, bl2):
    assert x.shape == (N, D) and edge_index.shape == (2, E)
    src = edge_index[0].astype(jnp.int32)
    dst = edge_index[1].astype(jnp.int32)
    bat3d = batch.astype(jnp.int32).reshape(N // BN, 1, BN)
    ea = edge_attr.astype(jnp.float32)

    z128 = jnp.zeros((N, D), jnp.float32)
    eye4 = jnp.eye(H, dtype=jnp.float32)
    rrep = jnp.repeat(eye4, D // H, axis=0)

    h = x
    psums = []
    pcnt = None
    for p in params:
        att = p['att']
        ahat = (eye4[:, None, :] * att[:, :, None]).reshape(D, H)
        m1 = ahat @ rrep.T
        gs = (p['gamma'] / jnp.sqrt(1.0 + 1e-5)).reshape(1, D)
        bt = p['beta'].reshape(1, D)
        epsv = p['eps'].reshape(1, 1)

        sc_gather, sc_scatter = _sc_kernels()
        xp = _tc_linear(h, p['Wx'], p['bx'], BN)
        g = sc_gather(xp, src)
        v128, vex = _tc_edge(g, ea, p['We'], p['be'], m1)
        p128 = sc_scatter(v128, dst, z128)
        pex = sc_scatter(vex, dst, z128)
        h, ps, pc = _tc_node(xp, p128, pex, bat3d,
                             p['W1'], p['b1'], gs, bt, p['W2'], p['b2'], epsv)
        psums.append(ps)
        if pcnt is None:
            pcnt = pc

    w1r = Wl1.reshape(5, D, 5 * D)
    w2p = jnp.pad(Wl2, ((0, 0), (0, D - 1)))
    out2d = _tc_head(psums, pcnt, w1r, bl1, w2p, bl2)
    return out2d[:, 0]


# speedup vs baseline: 6.8519x; 1.0301x over previous
import functools

import jax
import jax.numpy as jnp
from jax import lax
from jax.experimental import pallas as pl
from jax.experimental.pallas import tpu as pltpu
from jax.experimental.pallas import tpu_sc as plsc

N = 10000
E = 320000
D = 128
ED = 16
H = 4
NG = 64

NC = 2
NS = 16
NW = NC * NS
EW = E // NW
CH = 80
KG = 5
KGG = 4
GEG = KGG * CH
GPWG = (EW - CH) // GEG
CHS = 40
GES = KG * CHS
GPWS = EW // GES
RS = 624
RTAIL = N - NS * RS

@functools.cache
def _sc_kernels():
    mesh = plsc.VectorSubcoreMesh(core_axis_name="c", subcore_axis_name="s")

    @functools.partial(
        pl.kernel,
        mesh=mesh,
        out_type=jax.ShapeDtypeStruct((E, D), jnp.float32),
        scratch_types=[
            pltpu.VMEM((CH,), jnp.int32),
            pltpu.VMEM((CH,), jnp.int32),
            pltpu.VMEM((CH,), jnp.int32),
            pltpu.VMEM((CH,), jnp.int32),
            pltpu.VMEM((KGG * CH, D), jnp.float32),
            pltpu.VMEM_SHARED((N, D), jnp.float32),
            pltpu.SemaphoreType.DMA,
            pltpu.SemaphoreType.DMA,
        ],
    )
    def _sc_gather(table_hbm, idx_hbm, out_hbm, i0, i1, i2, i3,
                   rows_v, tab_sh, seml, semg):
        cid = lax.axis_index("c")
        sid = lax.axis_index("s")
        wid = sid * NC + cid
        ebase = wid * EW
        ibufs = (i0, i1, i2, i3)
        rb = pl.multiple_of(sid * RS, 8)

        pltpu.sync_copy(table_hbm.at[pl.ds(rb, RS)], tab_sh.at[pl.ds(rb, RS)])

        @pl.when(sid == 0)
        def _():
            tb = NS * RS
            pltpu.sync_copy(table_hbm.at[pl.ds(tb, RTAIL)],
                            tab_sh.at[pl.ds(tb, RTAIL)])

        plsc.subcore_barrier()

        def body(g, carry):
            base = pl.multiple_of(ebase + g * GEG, 8)
            hs = [pltpu.async_copy(
                idx_hbm.at[pl.ds(base + k * CH, CH)], ibufs[k], seml)
                for k in range(KGG)]
            for h in hs:
                h.wait()
            gs = [pltpu.async_copy(
                tab_sh.at[ibufs[k]],
                rows_v.at[pl.ds(k * CH, CH)], semg) for k in range(KGG)]
            for h in gs:
                h.wait()
            pltpu.sync_copy(rows_v, out_hbm.at[pl.ds(base, GEG)])
            return carry

        lax.fori_loop(0, GPWG, body, 0)

        tbase = pl.multiple_of(ebase + GPWG * GEG, 8)
        pltpu.sync_copy(idx_hbm.at[pl.ds(tbase, CH)], i0)
        pltpu.async_copy(tab_sh.at[i0], rows_v.at[pl.ds(0, CH)], semg).wait()
        pltpu.sync_copy(rows_v.at[pl.ds(0, CH)], out_hbm.at[pl.ds(tbase, CH)])

    @functools.partial(
        pl.kernel,
        mesh=mesh,
        out_type=jax.ShapeDtypeStruct((NC, N, D), jnp.float32),
        scratch_types=[
            pltpu.VMEM((CHS,), jnp.int32),
            pltpu.VMEM((CHS,), jnp.int32),
            pltpu.VMEM((CHS,), jnp.int32),
            pltpu.VMEM((CHS,), jnp.int32),
            pltpu.VMEM((CHS,), jnp.int32),
            pltpu.VMEM((GES, D), jnp.float32),
            pltpu.VMEM_SHARED((N, D), jnp.float32),
            pltpu.SemaphoreType.DMA,
            pltpu.SemaphoreType.DMA,
        ],
    )
    def _sc_scatter(v_hbm, dst_hbm, z_hbm, o_hbm, i0, i1, i2, i3, i4,
                    r_v, acc, seml, sems):
        cid = lax.axis_index("c")
        sid = lax.axis_index("s")
        wid = sid * NC + cid
        rb = pl.multiple_of(sid * RS, 8)
        ibufs = (i0, i1, i2, i3, i4)

        pltpu.sync_copy(z_hbm.at[pl.ds(rb, RS)], acc.at[pl.ds(rb, RS)])

        @pl.when(sid == 0)
        def _():
            tb = NS * RS
            pltpu.sync_copy(z_hbm.at[pl.ds(tb, RTAIL)], acc.at[pl.ds(tb, RTAIL)])

        plsc.subcore_barrier()

        ebase = wid * EW

        def body(g, carry):
            base = pl.multiple_of(ebase + g * GES, 8)
            hs = [pltpu.async_copy(
                dst_hbm.at[pl.ds(base + k * CHS, CHS)], ibufs[k], seml)
                for k in range(KG)]
            hs.append(pltpu.async_copy(
                v_hbm.at[pl.ds(base, GES)], r_v, seml))
            for h in hs:
                h.wait()
            ss = [pltpu.async_copy(
                r_v.at[pl.ds(k * CHS, CHS)],
                acc.at[ibufs[k]], sems, add=True) for k in range(KG)]
            for h in ss:
                h.wait()
            return carry

        lax.fori_loop(0, GPWS, body, 0)
        plsc.subcore_barrier()

        pltpu.sync_copy(acc.at[pl.ds(rb, RS)], o_hbm.at[cid, pl.ds(rb, RS)])

        @pl.when(sid == 0)
        def _():
            tb = NS * RS
            pltpu.sync_copy(acc.at[pl.ds(tb, RTAIL)],
                            o_hbm.at[cid, pl.ds(tb, RTAIL)])

    return _sc_gather, _sc_scatter


def _mm_body(x_ref, w_ref, b_ref, o_ref):
    o_ref[...] = (jnp.dot(x_ref[...], w_ref[...],
                          preferred_element_type=jnp.float32) + b_ref[...])


def _tc_linear(x, w, b, bn):
    n = x.shape[0]
    din, dout = w.shape
    return pl.pallas_call(
        _mm_body,
        grid=(n // bn,),
        in_specs=[
            pl.BlockSpec((bn, din), lambda i: (i, 0)),
            pl.BlockSpec((din, dout), lambda i: (0, 0)),
            pl.BlockSpec((1, dout), lambda i: (0, 0)),
        ],
        out_specs=pl.BlockSpec((bn, dout), lambda i: (i, 0)),
        out_shape=jax.ShapeDtypeStruct((n, dout), jnp.float32),
    )(x, w, b.reshape(1, dout))


BE = 2000


def _edge_body(g_ref, ea_ref, we_ref, be_ref, m1_ref, o128_ref, oex_ref):
    ep = jnp.dot(ea_ref[...], we_ref[...],
                 preferred_element_type=jnp.float32) + be_ref[...]
    m = jnp.maximum(g_ref[...].astype(jnp.float32) + ep, 0.0)
    exb = jnp.exp(jnp.dot(m, m1_ref[...], preferred_element_type=jnp.float32))
    o128_ref[...] = m * exb
    oex_ref[...] = exb


def _tc_edge(g, ea, we, be, m1):
    return pl.pallas_call(
        _edge_body,
        grid=(E // BE,),
        in_specs=[
            pl.BlockSpec((BE, D), lambda i: (i, 0)),
            pl.BlockSpec((BE, ED), lambda i: (i, 0)),
            pl.BlockSpec((ED, D), lambda i: (0, 0)),
            pl.BlockSpec((1, D), lambda i: (0, 0)),
            pl.BlockSpec((D, D), lambda i: (0, 0)),
        ],
        out_specs=[
            pl.BlockSpec((BE, D), lambda i: (i, 0)),
            pl.BlockSpec((BE, D), lambda i: (i, 0)),
        ],
        out_shape=[
            jax.ShapeDtypeStruct((E, D), jnp.float32),
            jax.ShapeDtypeStruct((E, D), jnp.float32),
        ],
    )(g, ea, we, be.reshape(1, D), m1)


BN = 1000


def _node_body(xp_ref, p0_ref, p1_ref, q0_ref, q1_ref, bat_ref,
               w1_ref, b1_ref, gs_ref, bt_ref, w2_ref, b2_ref, eps_ref,
               h_ref, ps_ref, pc_ref):
    den = q0_ref[...] + q1_ref[...] + 1e-16
    agg = (p0_ref[...] + p1_ref[...]) / den
    h1 = (1.0 + eps_ref[0, 0]) * xp_ref[...] + agg
    h2 = jnp.dot(h1, w1_ref[...], preferred_element_type=jnp.float32) + b1_ref[...]
    h2 = jnp.maximum(gs_ref[...] * h2 + bt_ref[...], 0.0)
    h3 = jnp.dot(h2, w2_ref[...], preferred_element_type=jnp.float32) + b2_ref[...]
    hout = jnp.maximum(h3, 0.0)
    h_ref[...] = hout

    g64 = lax.broadcasted_iota(jnp.int32, (NG, BN), 0)
    mask = (g64 == bat_ref[0]).astype(jnp.float32)
    s = jnp.dot(mask, hout, preferred_element_type=jnp.float32)
    c = jnp.dot(mask, jnp.ones((BN, D), jnp.float32),
                preferred_element_type=jnp.float32)

    i = pl.program_id(0)

    @pl.when(i == 0)
    def _():
        ps_ref[...] = s
        pc_ref[...] = c

    @pl.when(i > 0)
    def _():
        ps_ref[...] += s
        pc_ref[...] += c


def _tc_node(xp, p128, pex, bat3d, w1, b1, gs, bt, w2, b2, epsv):
    return pl.pallas_call(
        _node_body,
        grid=(N // BN,),
        in_specs=[
            pl.BlockSpec((BN, D), lambda i: (i, 0)),
            pl.BlockSpec((BN, D), lambda i: (i, 0)),
            pl.BlockSpec((BN, D), lambda i: (i, 0)),
            pl.BlockSpec((BN, D), lambda i: (i, 0)),
            pl.BlockSpec((BN, D), lambda i: (i, 0)),
            pl.BlockSpec((1, 1, BN), lambda i: (i, 0, 0)),
            pl.BlockSpec((D, D), lambda i: (0, 0)),
            pl.BlockSpec((1, D), lambda i: (0, 0)),
            pl.BlockSpec((1, D), lambda i: (0, 0)),
            pl.BlockSpec((1, D), lambda i: (0, 0)),
            pl.BlockSpec((D, D), lambda i: (0, 0)),
            pl.BlockSpec((1, D), lambda i: (0, 0)),
            pl.BlockSpec((1, 1), lambda i: (0, 0)),
        ],
        out_specs=[
            pl.BlockSpec((BN, D), lambda i: (i, 0)),
            pl.BlockSpec((NG, D), lambda i: (0, 0)),
            pl.BlockSpec((NG, D), lambda i: (0, 0)),
        ],
        out_shape=[
            jax.ShapeDtypeStruct((N, D), jnp.float32),
            jax.ShapeDtypeStruct((NG, D), jnp.float32),
            jax.ShapeDtypeStruct((NG, D), jnp.float32),
        ],
    )(xp, p128[0], p128[1], pex[0], pex[1], bat3d,
      w1, b1.reshape(1, D), gs, bt, w2, b2.reshape(1, D), epsv)


def _head_body(ps0_ref, ps1_ref, ps2_ref, ps3_ref, ps4_ref, pc_ref,
               w1r_ref, bl1_ref, w2p_ref, bl2_ref, o_ref):
    cnt = jnp.maximum(pc_ref[...], 1.0)
    acc = jnp.broadcast_to(bl1_ref[...], (NG, 5 * D))
    for l, ps in enumerate((ps0_ref, ps1_ref, ps2_ref, ps3_ref, ps4_ref)):
        pooled = ps[...] / cnt
        acc = acc + jnp.dot(pooled, w1r_ref[l],
                            preferred_element_type=jnp.float32)
    r = jnp.maximum(acc, 0.0)
    o_ref[...] = jnp.dot(r, w2p_ref[...],
                         preferred_element_type=jnp.float32) + bl2_ref[0, 0]


def _tc_head(psums, pcnt, w1r, bl1, w2p, bl2):
    return pl.pallas_call(
        _head_body,
        out_shape=jax.ShapeDtypeStruct((NG, D), jnp.float32),
    )(psums[0], psums[1], psums[2], psums[3], psums[4], pcnt,
      w1r, bl1.reshape(1, 5 * D), w2p, bl2.reshape(1, 1))


def kernel(x, edge_index, edge_attr, batch, params, Wl1, bl1, Wl2, bl2):
    assert x.shape == (N, D) and edge_index.shape == (2, E)
    src = edge_index[0].astype(jnp.int32)
    dst = edge_index[1].astype(jnp.int32)
    bat3d = batch.astype(jnp.int32).reshape(N // BN, 1, BN)
    ea = edge_attr.astype(jnp.float32)

    z128 = jnp.zeros((N, D), jnp.float32)
    eye4 = jnp.eye(H, dtype=jnp.float32)
    rrep = jnp.repeat(eye4, D // H, axis=0)

    h = x
    psums = []
    pcnt = None
    for p in params:
        att = p['att']
        ahat = (eye4[:, None, :] * att[:, :, None]).reshape(D, H)
        m1 = ahat @ rrep.T
        gs = (p['gamma'] / jnp.sqrt(1.0 + 1e-5)).reshape(1, D)
        bt = p['beta'].reshape(1, D)
        epsv = p['eps'].reshape(1, 1)

        sc_gather, sc_scatter = _sc_kernels()
        xp = _tc_linear(h, p['Wx'], p['bx'], BN)
        g = sc_gather(xp, src)
        v128, vex = _tc_edge(g, ea, p['We'], p['be'], m1)
        p128 = sc_scatter(v128, dst, z128)
        pex = sc_scatter(vex, dst, z128)
        h, ps, pc = _tc_node(xp, p128, pex, bat3d,
                             p['W1'], p['b1'], gs, bt, p['W2'], p['b2'], epsv)
        psums.append(ps)
        if pcnt is None:
            pcnt = pc

    w1r = Wl1.reshape(5, D, 5 * D)
    w2p = jnp.pad(Wl2, ((0, 0), (0, D - 1)))
    out2d = _tc_head(psums, pcnt, w1r, bl1, w2p, bl2)
    return out2d[:, 0]
